# baseline (speedup 1.0000x reference)
import functools

import jax
import jax.numpy as jnp
from jax import lax
from jax.experimental import pallas as pl
from jax.experimental.pallas import tpu as pltpu
from jax.experimental.pallas import tpu_sc as plsc

B, N, D, U = 128, 1024, 768, 1024
TB = 2
TM = TB * N
NTILES = B // TB
CU = 256
NCHUNK = U // CU


def _hvec_body(hid_ref, wh_ref, bias_ref, o_ref):
    o_ref[...] = (
        jnp.dot(hid_ref[...], wh_ref[...], preferred_element_type=jnp.float32)
        + bias_ref[...]
    )


def _hvec(hidden, Wh, bias2d):
    return pl.pallas_call(
        _hvec_body,
        out_shape=jax.ShapeDtypeStruct((B, U), jnp.float32),
    )(hidden, Wh, bias2d)


def _score_body(feat_ref, wf_ref, hb_ref, ws_ref, bs_ref,
                alpha_ref, idx_ref, lg_ref):
    i = pl.program_id(0)

    @pl.when(i < NTILES)
    def _score():
        x = feat_ref[0]
        lg128 = jnp.zeros((TM, 128), jnp.float32)
        for c in range(NCHUNK):
            sl = slice(c * CU, (c + 1) * CU)
            acc = jnp.dot(x, wf_ref[:, sl], preferred_element_type=jnp.float32)
            pre = acc.reshape(TB, N, CU) + hb_ref[:, :, sl]
            t = jnp.tanh(pre).reshape(TM, CU)
            lg128 = lg128 + jnp.dot(t, ws_ref[sl, :],
                                    preferred_element_type=jnp.float32)
        lg_ref[i % 2] = lg128

    @pl.when(i > 0)
    def _epilogue():
        lg128p = lg_ref[1 - (i % 2)]
        lg = jnp.sum(lg128p, axis=-1)
        x = lg.reshape(TB, N) + bs_ref[0]
        m = jnp.max(x, axis=1, keepdims=True)
        e = jnp.exp(x - m)
        s = jnp.sum(e, axis=1, keepdims=True)
        a = e / s
        alpha_ref[...] = a.reshape(TB, 1, N)
        am = jnp.max(a, axis=1, keepdims=True)
        col = lax.broadcasted_iota(jnp.int32, (TB, N), 1)
        locs = jnp.min(jnp.where(a == am, col, N), axis=1)
        boff = (lax.broadcasted_iota(jnp.int32, (TB, 1, 128), 0)
                + (i - 1) * TB) * N
        idx_ref[...] = boff + locs.reshape(TB, 1, 1)


def _scores(features3, Wf, hb3, ws_pad, bs):
    return pl.pallas_call(
        _score_body,
        grid=(NTILES + 1,),
        in_specs=[
            pl.BlockSpec((1, TM, D), lambda i: (jnp.minimum(i, NTILES - 1), 0, 0)),
            pl.BlockSpec((D, U), lambda i: (0, 0)),
            pl.BlockSpec((TB, 1, U), lambda i: (jnp.minimum(i, NTILES - 1), 0, 0)),
            pl.BlockSpec((U, 128), lambda i: (0, 0)),
            pl.BlockSpec(memory_space=pltpu.SMEM),
        ],
        out_specs=[
            pl.BlockSpec((TB, 1, N), lambda i: (jnp.maximum(i - 1, 0), 0, 0)),
            pl.BlockSpec((TB, 1, 128), lambda i: (jnp.maximum(i - 1, 0), 0, 0)),
        ],
        out_shape=[
            jax.ShapeDtypeStruct((B, 1, N), jnp.float32),
            jax.ShapeDtypeStruct((B, 1, 128), jnp.int32),
        ],
        scratch_shapes=[pltpu.VMEM((2, TM, 128), jnp.float32)],
    )(features3, Wf, hb3, ws_pad, bs)


_NWU = 16
_RPW = B // _NWU


def _make_gather():
    info = plsc.get_sparse_core_info()
    nc = info.num_cores
    mesh = plsc.VectorSubcoreMesh(core_axis_name="c", subcore_axis_name="s")

    @functools.partial(
        pl.kernel,
        mesh=mesh,
        out_type=jax.ShapeDtypeStruct((B, D), jnp.float32),
        scratch_types=[
            pltpu.VMEM((_RPW,), jnp.int32),
            pltpu.VMEM((_RPW, D), jnp.float32),
            pltpu.SemaphoreType.DMA,
        ],
    )
    def gather(table_hbm, idx_hbm, out_hbm, idx_v, rows_v, sem):
        wid = lax.axis_index("s") * nc + lax.axis_index("c")

        @pl.when(wid < _NWU)
        def _():
            base = wid * _RPW
            pltpu.sync_copy(idx_hbm.at[pl.ds(base, _RPW)], idx_v)
            pltpu.async_copy(table_hbm.at[idx_v], rows_v, sem).wait()
            pltpu.sync_copy(rows_v, out_hbm.at[pl.ds(base, _RPW)])

    return gather


_gather = _make_gather()


def kernel(features, hidden, Wf, bf, Wh, bh, Ws, bs):
    hvec = _hvec(hidden, Wh, bh.reshape(1, U)).reshape(B, 1, U)
    hb = hvec + bf.reshape(1, 1, U)
    ws_pad = jnp.zeros((U, 128), jnp.float32).at[:, 0].set(Ws[:, 0])
    features3 = features.reshape(NTILES, TM, D)
    alpha3, idx3 = _scores(features3, Wf, hb, ws_pad, bs)
    table = features.reshape(B * N, D)
    context = _gather(table, idx3[:, 0, 0])
    return (context, alpha3.reshape(B, N, 1))


# speedup vs baseline: 1.4538x; 1.4538x over previous
import functools

import jax
import jax.numpy as jnp
from jax import lax
from jax.experimental import pallas as pl
from jax.experimental.pallas import tpu as pltpu
from jax.experimental.pallas import tpu_sc as plsc

B, N, D, U = 128, 1024, 768, 1024
TILE_N = 512
NT = N // TILE_N


def _hvec_body(hid_ref, wh_ref, bias_ref, o_ref):
    o_ref[...] = (
        jnp.dot(hid_ref[...], wh_ref[...], preferred_element_type=jnp.float32)
        + bias_ref[...]
    )


def _hvec(hidden, Wh, bias2d):
    return pl.pallas_call(
        _hvec_body,
        out_shape=jax.ShapeDtypeStruct((B, U), jnp.float32),
    )(hidden, Wh, bias2d)


def _score_body(feat_ref, wf_ref, hb_ref, ws_ref, o_ref):
    x = feat_ref[0]
    acc = jnp.dot(x, wf_ref[...], preferred_element_type=jnp.float32)
    t = jnp.tanh(acc + hb_ref[0])
    logit = jnp.dot(t, ws_ref[...], preferred_element_type=jnp.float32)[:, 0]
    o_ref[0, 0, :] = logit


def _scores(features, Wf, hb, ws_pad):
    return pl.pallas_call(
        _score_body,
        grid=(B, NT),
        in_specs=[
            pl.BlockSpec((1, TILE_N, D), lambda b, n: (b, n, 0)),
            pl.BlockSpec((D, U), lambda b, n: (0, 0)),
            pl.BlockSpec((1, 1, U), lambda b, n: (b, 0, 0)),
            pl.BlockSpec((U, 128), lambda b, n: (0, 0)),
        ],
        out_specs=pl.BlockSpec((1, 1, TILE_N), lambda b, n: (b, 0, n)),
        out_shape=jax.ShapeDtypeStruct((B, 1, N), jnp.float32),
    )(features, Wf, hb, ws_pad)


def _softmax_body(logits_ref, bs_ref, alpha_ref, idx_ref):
    x = logits_ref[...] + bs_ref[0]
    m = jnp.max(x, axis=1, keepdims=True)
    e = jnp.exp(x - m)
    s = jnp.sum(e, axis=1, keepdims=True)
    a = e / s
    alpha_ref[...] = a
    am = jnp.max(a, axis=1, keepdims=True)
    col = lax.broadcasted_iota(jnp.int32, (B, N), 1)
    loc = jnp.min(jnp.where(a == am, col, N), axis=1)
    row0 = lax.broadcasted_iota(jnp.int32, (1, B), 1) * N
    idx_ref[...] = row0 + loc[None, :]


def _softmax_argmax(logits, bs):
    return pl.pallas_call(
        _softmax_body,
        in_specs=[
            pl.BlockSpec((B, N), lambda: (0, 0)),
            pl.BlockSpec(memory_space=pltpu.SMEM),
        ],
        out_specs=[
            pl.BlockSpec((B, N), lambda: (0, 0)),
            pl.BlockSpec((1, B), lambda: (0, 0)),
        ],
        out_shape=[
            jax.ShapeDtypeStruct((B, N), jnp.float32),
            jax.ShapeDtypeStruct((1, B), jnp.int32),
        ],
    )(logits, bs)


_NWU = 16
_RPW = B // _NWU


def _make_gather():
    info = plsc.get_sparse_core_info()
    nc = info.num_cores
    mesh = plsc.VectorSubcoreMesh(core_axis_name="c", subcore_axis_name="s")

    @functools.partial(
        pl.kernel,
        mesh=mesh,
        out_type=jax.ShapeDtypeStruct((B, D), jnp.float32),
        scratch_types=[
            pltpu.VMEM((_RPW,), jnp.int32),
            pltpu.VMEM((_RPW, D), jnp.float32),
            pltpu.SemaphoreType.DMA,
        ],
    )
    def gather(table_hbm, idx_hbm, out_hbm, idx_v, rows_v, sem):
        wid = lax.axis_index("s") * nc + lax.axis_index("c")

        @pl.when(wid < _NWU)
        def _():
            base = wid * _RPW
            pltpu.sync_copy(idx_hbm.at[pl.ds(base, _RPW)], idx_v)
            pltpu.async_copy(table_hbm.at[idx_v], rows_v, sem).wait()
            pltpu.sync_copy(rows_v, out_hbm.at[pl.ds(base, _RPW)])

    return gather


_gather = _make_gather()


def kernel(features, hidden, Wf, bf, Wh, bh, Ws, bs):
    hvec = _hvec(hidden, Wh, bh.reshape(1, U)).reshape(B, 1, U)
    hb = hvec + bf.reshape(1, 1, U)
    ws_pad = jnp.zeros((U, 128), jnp.float32).at[:, 0].set(Ws[:, 0])
    logits = _scores(features, Wf, hb, ws_pad)
    alpha2d, idx2d = _softmax_argmax(logits.reshape(B, N), bs)
    table = features.reshape(B * N, D)
    context = _gather(table, idx2d.reshape(B))
    return (context, alpha2d.reshape(B, N, 1))
